# Initial kernel scaffold; baseline (speedup 1.0000x reference)
import jax
import jax.numpy as jnp
from jax.experimental import pallas as pl


def kernel(nfeats, efeats, edge_index, Wm1, bm1, Wa1, ba1, Wm2, bm2, Wa2, ba2, Wm3, bm3, Wa3, ba3, Wp, bp):
    raise NotImplementedError("write your pallas kernel here")


# speedup vs baseline: 4.8291x; 4.8291x over previous
import functools
import jax
import jax.numpy as jnp
from jax import lax
from jax.experimental import pallas as pl
from jax.experimental.pallas import tpu as pltpu
from jax.experimental.pallas import tpu_sc as plsc

_NS = 16
_NC = 2
_K = 80
_BR = 2000


def _pad_cols(w, width):
    return jnp.pad(w, ((0, 0), (0, width - w.shape[1])))


def _pad_rows(w, height):
    return jnp.pad(w, ((0, height - w.shape[0]), (0, 0)))


def _tc_pre(h0, wpre, n, hp):
    din = h0.shape[1]
    r = n // _BR

    def body(h_ref, w_ref, t_ref):
        t_ref[...] = jnp.dot(h_ref[...], w_ref[0],
                             preferred_element_type=jnp.float32)

    return pl.pallas_call(
        body,
        grid=(r, _NC),
        in_specs=[
            pl.BlockSpec((_BR, din), lambda i, c: (i, 0)),
            pl.BlockSpec((1, din, hp), lambda i, c: (c, 0, 0)),
        ],
        out_specs=pl.BlockSpec((_BR, hp), lambda i, c: (c * r + i, 0)),
        out_shape=jax.ShapeDtypeStruct((2 * n, hp), jnp.float32),
    )(h0, wpre)


def _tc_mid(h, s2, e2, wmb0, wmb1, waa, wab0, wab1, ba2, wnext, n, hp, emb):
    dh = h.shape[1]
    r = n // _BR

    def body(h_ref, s0_ref, s1_ref, e0_ref, e1_ref, wmb0_ref, wmb1_ref,
             waa_ref, wab0_ref, wab1_ref, ba_ref, wn_ref, hout_ref, tout_ref):
        ea = e0_ref[...] + e1_ref[...]
        inv = 1.0 / jnp.maximum(ea[:, 0:1], 1.0)
        hn0 = (s0_ref[...] + jnp.dot(ea, wmb0_ref[...],
                                     preferred_element_type=jnp.float32)) * inv
        hn1 = (s1_ref[...] + jnp.dot(ea, wmb1_ref[...],
                                     preferred_element_type=jnp.float32)) * inv
        hnew = jnp.maximum(
            jnp.dot(h_ref[...], waa_ref[...],
                    preferred_element_type=jnp.float32)
            + jnp.dot(hn0, wab0_ref[...], preferred_element_type=jnp.float32)
            + jnp.dot(hn1, wab1_ref[...], preferred_element_type=jnp.float32)
            + ba_ref[...], 0.0)
        hout_ref[...] = hnew
        tout_ref[...] = jnp.dot(hnew, wn_ref[0],
                                preferred_element_type=jnp.float32)

    full = lambda shape: pl.BlockSpec(shape, lambda i, c: tuple(0 for _ in shape))
    return pl.pallas_call(
        body,
        grid=(r, _NC),
        in_specs=[
            pl.BlockSpec((_BR, dh), lambda i, c: (i, 0)),
            pl.BlockSpec((_BR, hp), lambda i, c: (i, 0)),
            pl.BlockSpec((_BR, hp), lambda i, c: (r + i, 0)),
            pl.BlockSpec((_BR, hp), lambda i, c: (i, 0)),
            pl.BlockSpec((_BR, hp), lambda i, c: (r + i, 0)),
            full((hp, hp)),
            full((hp, hp)),
            full((dh, emb)),
            full((hp, emb)),
            full((hp, emb)),
            full((1, emb)),
            pl.BlockSpec((1, emb, hp), lambda i, c: (c, 0, 0)),
        ],
        out_specs=[
            pl.BlockSpec((_BR, emb), lambda i, c: (i, 0)),
            pl.BlockSpec((_BR, hp), lambda i, c: (c * r + i, 0)),
        ],
        out_shape=[
            jax.ShapeDtypeStruct((n, emb), jnp.float32),
            jax.ShapeDtypeStruct((2 * n, hp), jnp.float32),
        ],
    )(h, s2, s2, e2, e2, wmb0, wmb1, waa, wab0, wab1, ba2, wnext)


def _tc_final(h, s2, e2, wmb0, wmb1, waa, wab0, wab1, ba2, wp_pad, bp_pad,
              n, hp, emb):
    dh = h.shape[1]
    r = n // _BR

    def body(h_ref, s0_ref, s1_ref, e0_ref, e1_ref, wmb0_ref, wmb1_ref,
             waa_ref, wab0_ref, wab1_ref, ba_ref, wp_ref, bp_ref, ab_ref):
        ea = e0_ref[...] + e1_ref[...]
        inv = 1.0 / jnp.maximum(ea[:, 0:1], 1.0)
        hn0 = (s0_ref[...] + jnp.dot(ea, wmb0_ref[...],
                                     preferred_element_type=jnp.float32)) * inv
        hn1 = (s1_ref[...] + jnp.dot(ea, wmb1_ref[...],
                                     preferred_element_type=jnp.float32)) * inv
        hnew = jnp.maximum(
            jnp.dot(h_ref[...], waa_ref[...],
                    preferred_element_type=jnp.float32)
            + jnp.dot(hn0, wab0_ref[...], preferred_element_type=jnp.float32)
            + jnp.dot(hn1, wab1_ref[...], preferred_element_type=jnp.float32)
            + ba_ref[...], 0.0)
        ab_ref[...] = jnp.dot(hnew, wp_ref[...],
                              preferred_element_type=jnp.float32) + bp_ref[...]

    full = lambda shape: pl.BlockSpec(shape, lambda i: tuple(0 for _ in shape))
    return pl.pallas_call(
        body,
        grid=(r,),
        in_specs=[
            pl.BlockSpec((_BR, dh), lambda i: (i, 0)),
            pl.BlockSpec((_BR, hp), lambda i: (i, 0)),
            pl.BlockSpec((_BR, hp), lambda i: (r + i, 0)),
            pl.BlockSpec((_BR, hp), lambda i: (i, 0)),
            pl.BlockSpec((_BR, hp), lambda i: (r + i, 0)),
            full((hp, hp)),
            full((hp, hp)),
            full((dh, emb)),
            full((hp, emb)),
            full((hp, emb)),
            full((1, emb)),
            full((emb, 128)),
            full((1, 128)),
        ],
        out_specs=pl.BlockSpec((_BR, 128), lambda i: (i, 0)),
        out_shape=jax.ShapeDtypeStruct((n, 128), jnp.float32),
    )(h, s2, s2, e2, e2, wmb0, wmb1, waa, wab0, wab1, ba2, wp_pad, bp_pad)


def _sc_scatter(tbl, srcr, dstr, zer, n, hp, nchunks):
    mesh = plsc.VectorSubcoreMesh(core_axis_name="c", subcore_axis_name="s")
    npr = (n // _NS) // 8 * 8
    tail = n - _NS * npr

    @functools.partial(
        pl.kernel,
        out_type=jax.ShapeDtypeStruct((2 * n, hp), jnp.float32),
        mesh=mesh,
        compiler_params=pltpu.CompilerParams(use_tc_tiling_on_sc=False, needs_layout_passes=False),
        scratch_types=[
            pltpu.VMEM((nchunks, _K), jnp.int32),
            pltpu.VMEM((nchunks, _K), jnp.int32),
            pltpu.VMEM((_K, hp), jnp.float32),
            pltpu.VMEM_SHARED((n, hp), jnp.float32),
            pltpu.SemaphoreType.DMA,
        ],
    )
    def body(tbl_ref, src_ref, dst_ref, zer_ref, out_ref,
             src_v, dst_v, rows_v, acc, sem):
        c = lax.axis_index("c")
        s = lax.axis_index("s")
        pltpu.sync_copy(src_ref.at[c, s], src_v)
        pltpu.sync_copy(dst_ref.at[s], dst_v)
        pltpu.sync_copy(zer_ref.at[pl.ds(s * npr, npr)],
                        acc.at[pl.ds(s * npr, npr)])
        if tail:
            @pl.when(s == 0)
            def _():
                pltpu.sync_copy(zer_ref.at[pl.ds(_NS * npr, tail)],
                                acc.at[pl.ds(_NS * npr, tail)])
        plsc.subcore_barrier()

        def chunk(j, carry):
            pltpu.async_copy(tbl_ref.at[src_v.at[j]], rows_v, sem).wait()
            pltpu.sync_copy(rows_v, acc.at[dst_v.at[j]], add=True)
            return carry

        lax.fori_loop(0, nchunks, chunk, 0)
        plsc.subcore_barrier()
        pltpu.sync_copy(acc.at[pl.ds(s * npr, npr)],
                        out_ref.at[pl.ds(c * n + s * npr, npr)])
        if tail:
            @pl.when(s == 0)
            def _():
                pltpu.sync_copy(acc.at[pl.ds(_NS * npr, tail)],
                                out_ref.at[pl.ds(c * n + _NS * npr, tail)])

    return body(tbl, srcr, dstr, zer)


def _sc_esum(ef_aug, dstr2, zer, n, hp, nchunks):
    mesh = plsc.VectorSubcoreMesh(core_axis_name="c", subcore_axis_name="s")
    npr = (n // _NS) // 8 * 8
    tail = n - _NS * npr
    ept = nchunks * _K

    @functools.partial(
        pl.kernel,
        out_type=jax.ShapeDtypeStruct((2 * n, hp), jnp.float32),
        mesh=mesh,
        compiler_params=pltpu.CompilerParams(use_tc_tiling_on_sc=False, needs_layout_passes=False),
        scratch_types=[
            pltpu.VMEM((nchunks, _K), jnp.int32),
            pltpu.VMEM((_K, hp), jnp.float32),
            pltpu.VMEM_SHARED((n, hp), jnp.float32),
        ],
    )
    def body(ef_ref, dst_ref, zer_ref, out_ref, dst_v, rows_v, acc):
        c = lax.axis_index("c")
        s = lax.axis_index("s")
        base = (c * _NS + s) * ept
        pltpu.sync_copy(dst_ref.at[c, s], dst_v)
        pltpu.sync_copy(zer_ref.at[pl.ds(s * npr, npr)],
                        acc.at[pl.ds(s * npr, npr)])
        if tail:
            @pl.when(s == 0)
            def _():
                pltpu.sync_copy(zer_ref.at[pl.ds(_NS * npr, tail)],
                                acc.at[pl.ds(_NS * npr, tail)])
        plsc.subcore_barrier()

        def chunk(j, carry):
            pltpu.sync_copy(ef_ref.at[pl.ds(base + j * _K, _K)], rows_v)
            pltpu.sync_copy(rows_v, acc.at[dst_v.at[j]], add=True)
            return carry

        lax.fori_loop(0, nchunks, chunk, 0)
        plsc.subcore_barrier()
        pltpu.sync_copy(acc.at[pl.ds(s * npr, npr)],
                        out_ref.at[pl.ds(c * n + s * npr, npr)])
        if tail:
            @pl.when(s == 0)
            def _():
                pltpu.sync_copy(acc.at[pl.ds(_NS * npr, tail)],
                                out_ref.at[pl.ds(c * n + _NS * npr, tail)])

    return body(ef_aug, dstr2, zer)


def _sc_final(ab4, src, dst, n, e):
    mesh = plsc.VectorSubcoreMesh(core_axis_name="c", subcore_axis_name="s")
    ept = e // (_NC * _NS)

    @functools.partial(
        pl.kernel,
        out_type=jax.ShapeDtypeStruct((2 * e,), jnp.float32),
        mesh=mesh,
        compiler_params=pltpu.CompilerParams(use_tc_tiling_on_sc=False, needs_layout_passes=False),
        scratch_types=[
            pltpu.VMEM((4 * n,), jnp.float32),
            pltpu.VMEM((ept,), jnp.int32),
            pltpu.VMEM((ept,), jnp.int32),
            pltpu.VMEM((2 * ept,), jnp.float32),
        ],
    )
    def body(ab_ref, src_ref, dst_ref, out_ref, ab_v, src_v, dst_v, out_v):
        c = lax.axis_index("c")
        s = lax.axis_index("s")
        w = c * _NS + s
        pltpu.sync_copy(ab_ref, ab_v)
        pltpu.sync_copy(src_ref.at[pl.ds(w * ept, ept)], src_v)
        pltpu.sync_copy(dst_ref.at[pl.ds(w * ept, ept)], dst_v)
        lanes = lax.iota(jnp.int32, 16)

        def step(i, carry):
            s16 = src_v[pl.ds(i * 16, 16)] * 4
            d16 = dst_v[pl.ds(i * 16, 16)] * 4
            a0 = plsc.load_gather(ab_v, [s16])
            a1 = plsc.load_gather(ab_v, [s16 + 1])
            b0 = plsc.load_gather(ab_v, [d16 + 2])
            b1 = plsc.load_gather(ab_v, [d16 + 3])
            rows = i * 32 + lanes * 2
            plsc.store_scatter(out_v, [rows], a0 + b0)
            plsc.store_scatter(out_v, [rows + 1], a1 + b1)
            return carry

        lax.fori_loop(0, ept // 16, step, 0)
        pltpu.sync_copy(out_v, out_ref.at[pl.ds(w * 2 * ept, 2 * ept)])

    return body(ab4, src, dst)


def kernel(nfeats, efeats, edge_index, Wm1, bm1, Wa1, ba1, Wm2, bm2, Wa2,
           ba2, Wm3, bm3, Wa3, ba3, Wp, bp):
    n = nfeats.shape[0]
    e = efeats.shape[0]
    din = nfeats.shape[2]
    edim = efeats.shape[2]
    emb = Wm1.shape[1]
    half = emb // 2
    hp = 80
    ncls = Wp.shape[1]

    src = edge_index[0]
    dst = edge_index[1]
    h0 = nfeats[:, 0, :]

    ef_aug = jnp.concatenate(
        [jnp.ones((e, 1), jnp.float32), efeats[:, 0, :],
         jnp.zeros((e, hp - 1 - edim), jnp.float32)], axis=1)

    def msg_weights(Wm, bm):
        dh = Wm.shape[0] - edim
        aug = jnp.concatenate(
            [bm[None, :], Wm[dh:], jnp.zeros((hp - 1 - edim, emb))], axis=0)
        return (_pad_cols(aug[:, :half], hp), _pad_cols(aug[:, half:], hp))

    def apply_weights(Wa, dh):
        waa = Wa[:dh]
        wab0 = _pad_rows(Wa[dh:dh + half], hp)
        wab1 = _pad_rows(Wa[dh + half:], hp)
        return waa, wab0, wab1

    def gather_table_w(Wm, dh):
        wa = Wm[:dh]
        return jnp.stack([_pad_cols(wa[:, :half], hp),
                          _pad_cols(wa[:, half:], hp)])

    wmb0_1, wmb1_1 = msg_weights(Wm1, bm1)
    wmb0_2, wmb1_2 = msg_weights(Wm2, bm2)
    wmb0_3, wmb1_3 = msg_weights(Wm3, bm3)
    waa1, wab0_1, wab1_1 = apply_weights(Wa1, din)
    waa2, wab0_2, wab1_2 = apply_weights(Wa2, emb)
    waa3, wab0_3, wab1_3 = apply_weights(Wa3, emb)
    wpre = gather_table_w(Wm1, din)
    wnext2 = gather_table_w(Wm2, emb)
    wnext3 = gather_table_w(Wm3, emb)
    wp_pad = _pad_cols(jnp.concatenate([Wp[:emb], Wp[emb:]], axis=1), 128)
    bp_pad = _pad_cols(bp[None, :], 128)

    eps = e // _NS
    cs = eps // _K
    ept = e // (_NC * _NS)
    ce = ept // _K
    srcr = (src.reshape(_NS, cs, _K)[None]
            + (jnp.arange(_NC, dtype=jnp.int32) * n)[:, None, None, None])
    dstr = dst.reshape(_NS, cs, _K)
    dstr2 = dst.reshape(_NC, _NS, ce, _K)
    zer = jnp.zeros((n, hp), jnp.float32)

    esum2 = _sc_esum(ef_aug, dstr2, zer, n, hp, ce)

    t1 = _tc_pre(h0, wpre, n, hp)
    s1 = _sc_scatter(t1, srcr, dstr, zer, n, hp, cs)
    h1, t2 = _tc_mid(h0, s1, esum2, wmb0_1, wmb1_1, waa1, wab0_1, wab1_1,
                     ba1[None, :], wnext2, n, hp, emb)
    s2 = _sc_scatter(t2, srcr, dstr, zer, n, hp, cs)
    h2, t3 = _tc_mid(h1, s2, esum2, wmb0_2, wmb1_2, waa2, wab0_2, wab1_2,
                     ba2[None, :], wnext3, n, hp, emb)
    s3 = _sc_scatter(t3, srcr, dstr, zer, n, hp, cs)
    ab = _tc_final(h2, s3, esum2, wmb0_3, wmb1_3, waa3, wab0_3, wab1_3,
                   ba3[None, :], wp_pad, bp_pad, n, hp, emb)

    ab4 = ab[:, :2 * ncls].reshape(2 * ncls * n)
    score = _sc_final(ab4, src, dst, n, e)
    return score.reshape(e, ncls)


# speedup vs baseline: 6.1079x; 1.2648x over previous
import functools
import jax
import jax.numpy as jnp
from jax import lax
from jax.experimental import pallas as pl
from jax.experimental.pallas import tpu as pltpu
from jax.experimental.pallas import tpu_sc as plsc

_NS = 16
_NC = 2
_K = 80
_BR = 2000


def _pad_cols(w, width):
    return jnp.pad(w, ((0, 0), (0, width - w.shape[1])))


def _pad_rows(w, height):
    return jnp.pad(w, ((0, height - w.shape[0]), (0, 0)))


def _tc_pre(h0, wpre, n, hp):
    din = h0.shape[1]
    r = n // _BR

    def body(h_ref, w_ref, t_ref):
        t_ref[...] = jnp.dot(h_ref[...], w_ref[0],
                             preferred_element_type=jnp.float32)

    return pl.pallas_call(
        body,
        grid=(r, _NC),
        in_specs=[
            pl.BlockSpec((_BR, din), lambda i, c: (i, 0)),
            pl.BlockSpec((1, din, hp), lambda i, c: (c, 0, 0)),
        ],
        out_specs=pl.BlockSpec((_BR, hp), lambda i, c: (c * r + i, 0)),
        out_shape=jax.ShapeDtypeStruct((2 * n, hp), jnp.float32),
    )(h0, wpre)


def _tc_mid(h, s2, e2, wmb0, wmb1, waa, wab0, wab1, ba2, wnext, n, hp, emb):
    dh = h.shape[1]
    r = n // _BR

    def body(h_ref, s0_ref, s1_ref, e0_ref, e1_ref, wmb0_ref, wmb1_ref,
             waa_ref, wab0_ref, wab1_ref, ba_ref, wn_ref, hout_ref, tout_ref):
        ea = e0_ref[...] + e1_ref[...]
        inv = 1.0 / jnp.maximum(ea[:, 0:1], 1.0)
        hn0 = (s0_ref[...] + jnp.dot(ea, wmb0_ref[...],
                                     preferred_element_type=jnp.float32)) * inv
        hn1 = (s1_ref[...] + jnp.dot(ea, wmb1_ref[...],
                                     preferred_element_type=jnp.float32)) * inv
        hnew = jnp.maximum(
            jnp.dot(h_ref[...], waa_ref[...],
                    preferred_element_type=jnp.float32)
            + jnp.dot(hn0, wab0_ref[...], preferred_element_type=jnp.float32)
            + jnp.dot(hn1, wab1_ref[...], preferred_element_type=jnp.float32)
            + ba_ref[...], 0.0)
        hout_ref[...] = hnew
        tout_ref[...] = jnp.dot(hnew, wn_ref[0],
                                preferred_element_type=jnp.float32)

    full = lambda shape: pl.BlockSpec(shape, lambda i, c: tuple(0 for _ in shape))
    return pl.pallas_call(
        body,
        grid=(r, _NC),
        in_specs=[
            pl.BlockSpec((_BR, dh), lambda i, c: (i, 0)),
            pl.BlockSpec((_BR, hp), lambda i, c: (i, 0)),
            pl.BlockSpec((_BR, hp), lambda i, c: (r + i, 0)),
            pl.BlockSpec((_BR, hp), lambda i, c: (i, 0)),
            pl.BlockSpec((_BR, hp), lambda i, c: (r + i, 0)),
            full((hp, hp)),
            full((hp, hp)),
            full((dh, emb)),
            full((hp, emb)),
            full((hp, emb)),
            full((1, emb)),
            pl.BlockSpec((1, emb, hp), lambda i, c: (c, 0, 0)),
        ],
        out_specs=[
            pl.BlockSpec((_BR, emb), lambda i, c: (i, 0)),
            pl.BlockSpec((_BR, hp), lambda i, c: (c * r + i, 0)),
        ],
        out_shape=[
            jax.ShapeDtypeStruct((n, emb), jnp.float32),
            jax.ShapeDtypeStruct((2 * n, hp), jnp.float32),
        ],
    )(h, s2, s2, e2, e2, wmb0, wmb1, waa, wab0, wab1, ba2, wnext)


def _tc_final(h, s2, e2, wmb0, wmb1, waa, wab0, wab1, ba2, wp_pad, bp_pad,
              n, hp, emb):
    dh = h.shape[1]
    r = n // _BR

    def body(h_ref, s0_ref, s1_ref, e0_ref, e1_ref, wmb0_ref, wmb1_ref,
             waa_ref, wab0_ref, wab1_ref, ba_ref, wp_ref, bp_ref, ab_ref):
        ea = e0_ref[...] + e1_ref[...]
        inv = 1.0 / jnp.maximum(ea[:, 0:1], 1.0)
        hn0 = (s0_ref[...] + jnp.dot(ea, wmb0_ref[...],
                                     preferred_element_type=jnp.float32)) * inv
        hn1 = (s1_ref[...] + jnp.dot(ea, wmb1_ref[...],
                                     preferred_element_type=jnp.float32)) * inv
        hnew = jnp.maximum(
            jnp.dot(h_ref[...], waa_ref[...],
                    preferred_element_type=jnp.float32)
            + jnp.dot(hn0, wab0_ref[...], preferred_element_type=jnp.float32)
            + jnp.dot(hn1, wab1_ref[...], preferred_element_type=jnp.float32)
            + ba_ref[...], 0.0)
        ab_ref[...] = jnp.dot(hnew, wp_ref[...],
                              preferred_element_type=jnp.float32) + bp_ref[...]

    full = lambda shape: pl.BlockSpec(shape, lambda i: tuple(0 for _ in shape))
    return pl.pallas_call(
        body,
        grid=(r,),
        in_specs=[
            pl.BlockSpec((_BR, dh), lambda i: (i, 0)),
            pl.BlockSpec((_BR, hp), lambda i: (i, 0)),
            pl.BlockSpec((_BR, hp), lambda i: (r + i, 0)),
            pl.BlockSpec((_BR, hp), lambda i: (i, 0)),
            pl.BlockSpec((_BR, hp), lambda i: (r + i, 0)),
            full((hp, hp)),
            full((hp, hp)),
            full((dh, emb)),
            full((hp, emb)),
            full((hp, emb)),
            full((1, emb)),
            full((emb, 128)),
            full((1, 128)),
        ],
        out_specs=pl.BlockSpec((_BR, 128), lambda i: (i, 0)),
        out_shape=jax.ShapeDtypeStruct((n, 128), jnp.float32),
    )(h, s2, s2, e2, e2, wmb0, wmb1, waa, wab0, wab1, ba2, wp_pad, bp_pad)


def _sc_scatter(tbl, srcr, dstr, zer, n, hp, nchunks):
    mesh = plsc.VectorSubcoreMesh(core_axis_name="c", subcore_axis_name="s")
    npr = (n // _NS) // 8 * 8
    tail = n - _NS * npr

    @functools.partial(
        pl.kernel,
        out_type=jax.ShapeDtypeStruct((2 * n, hp), jnp.float32),
        mesh=mesh,
        compiler_params=pltpu.CompilerParams(use_tc_tiling_on_sc=False, needs_layout_passes=False),
        scratch_types=[
            pltpu.VMEM((nchunks, _K), jnp.int32),
            pltpu.VMEM((nchunks, _K), jnp.int32),
            pltpu.VMEM((_K, hp), jnp.float32),
            pltpu.VMEM((_K, hp), jnp.float32),
            pltpu.VMEM_SHARED((n, hp), jnp.float32),
            pltpu.SemaphoreType.DMA,
            pltpu.SemaphoreType.DMA,
            pltpu.SemaphoreType.DMA,
            pltpu.SemaphoreType.DMA,
        ],
    )
    def body(tbl_ref, src_ref, dst_ref, zer_ref, out_ref,
             src_v, dst_v, rows0, rows1, acc, gs0, gs1, ss0, ss1):
        c = lax.axis_index("c")
        s = lax.axis_index("s")
        pltpu.sync_copy(src_ref.at[c, s], src_v)
        pltpu.sync_copy(dst_ref.at[s], dst_v)
        pltpu.sync_copy(zer_ref.at[pl.ds(s * npr, npr)],
                        acc.at[pl.ds(s * npr, npr)])
        if tail:
            @pl.when(s == 0)
            def _():
                pltpu.sync_copy(zer_ref.at[pl.ds(_NS * npr, tail)],
                                acc.at[pl.ds(_NS * npr, tail)])
        plsc.subcore_barrier()

        def g_issue(j, buf, sem):
            pltpu.async_copy(tbl_ref.at[src_v.at[j]], buf, sem)

        def g_wait(j, buf, sem):
            pltpu.make_async_copy(tbl_ref.at[src_v.at[j]], buf, sem).wait()

        def s_issue(j, buf, sem):
            pltpu.async_copy(buf, acc.at[dst_v.at[j]], sem, add=True)

        def s_wait(j, buf, sem):
            pltpu.make_async_copy(buf, acc.at[dst_v.at[j]], sem).wait()

        nh = nchunks // 2
        g_issue(0, rows0, gs0)
        g_issue(1, rows1, gs1)

        def pair(t, carry):
            g_wait(2 * t, rows0, gs0)
            s_issue(2 * t, rows0, ss0)
            g_wait(2 * t + 1, rows1, gs1)
            s_issue(2 * t + 1, rows1, ss1)

            @pl.when(t < nh - 1)
            def _():
                s_wait(2 * t, rows0, ss0)
                g_issue(2 * t + 2, rows0, gs0)
                s_wait(2 * t + 1, rows1, ss1)
                g_issue(2 * t + 3, rows1, gs1)
            return carry

        lax.fori_loop(0, nh, pair, 0)
        s_wait(2 * nh - 2, rows0, ss0)
        s_wait(2 * nh - 1, rows1, ss1)
        if nchunks % 2:
            j = nchunks - 1
            g_issue(j, rows0, gs0)
            g_wait(j, rows0, gs0)
            s_issue(j, rows0, ss0)
            s_wait(j, rows0, ss0)
        plsc.subcore_barrier()
        pltpu.sync_copy(acc.at[pl.ds(s * npr, npr)],
                        out_ref.at[pl.ds(c * n + s * npr, npr)])
        if tail:
            @pl.when(s == 0)
            def _():
                pltpu.sync_copy(acc.at[pl.ds(_NS * npr, tail)],
                                out_ref.at[pl.ds(c * n + _NS * npr, tail)])

    return body(tbl, srcr, dstr, zer)


def _sc_esum(ef_aug, dstr2, zer, n, hp, nchunks):
    mesh = plsc.VectorSubcoreMesh(core_axis_name="c", subcore_axis_name="s")
    npr = (n // _NS) // 8 * 8
    tail = n - _NS * npr
    ept = nchunks * _K

    @functools.partial(
        pl.kernel,
        out_type=jax.ShapeDtypeStruct((2 * n, hp), jnp.float32),
        mesh=mesh,
        compiler_params=pltpu.CompilerParams(use_tc_tiling_on_sc=False, needs_layout_passes=False),
        scratch_types=[
            pltpu.VMEM((nchunks, _K), jnp.int32),
            pltpu.VMEM((_K, hp), jnp.float32),
            pltpu.VMEM((_K, hp), jnp.float32),
            pltpu.VMEM_SHARED((n, hp), jnp.float32),
            pltpu.SemaphoreType.DMA,
            pltpu.SemaphoreType.DMA,
            pltpu.SemaphoreType.DMA,
            pltpu.SemaphoreType.DMA,
        ],
    )
    def body(ef_ref, dst_ref, zer_ref, out_ref, dst_v, rows0, rows1, acc,
             gs0, gs1, ss0, ss1):
        c = lax.axis_index("c")
        s = lax.axis_index("s")
        base = (c * _NS + s) * ept
        pltpu.sync_copy(dst_ref.at[c, s], dst_v)
        pltpu.sync_copy(zer_ref.at[pl.ds(s * npr, npr)],
                        acc.at[pl.ds(s * npr, npr)])
        if tail:
            @pl.when(s == 0)
            def _():
                pltpu.sync_copy(zer_ref.at[pl.ds(_NS * npr, tail)],
                                acc.at[pl.ds(_NS * npr, tail)])
        plsc.subcore_barrier()

        def g_issue(j, buf, sem):
            pltpu.async_copy(ef_ref.at[pl.ds(base + j * _K, _K)], buf, sem)

        def g_wait(j, buf, sem):
            pltpu.make_async_copy(ef_ref.at[pl.ds(base + j * _K, _K)],
                                  buf, sem).wait()

        def s_issue(j, buf, sem):
            pltpu.async_copy(buf, acc.at[dst_v.at[j]], sem, add=True)

        def s_wait(j, buf, sem):
            pltpu.make_async_copy(buf, acc.at[dst_v.at[j]], sem).wait()

        nh = nchunks // 2
        g_issue(0, rows0, gs0)
        g_issue(1, rows1, gs1)

        def pair(t, carry):
            g_wait(2 * t, rows0, gs0)
            s_issue(2 * t, rows0, ss0)
            g_wait(2 * t + 1, rows1, gs1)
            s_issue(2 * t + 1, rows1, ss1)

            @pl.when(t < nh - 1)
            def _():
                s_wait(2 * t, rows0, ss0)
                g_issue(2 * t + 2, rows0, gs0)
                s_wait(2 * t + 1, rows1, ss1)
                g_issue(2 * t + 3, rows1, gs1)
            return carry

        lax.fori_loop(0, nh, pair, 0)
        s_wait(2 * nh - 2, rows0, ss0)
        s_wait(2 * nh - 1, rows1, ss1)
        if nchunks % 2:
            j = nchunks - 1
            g_issue(j, rows0, gs0)
            g_wait(j, rows0, gs0)
            s_issue(j, rows0, ss0)
            s_wait(j, rows0, ss0)
        plsc.subcore_barrier()
        pltpu.sync_copy(acc.at[pl.ds(s * npr, npr)],
                        out_ref.at[pl.ds(c * n + s * npr, npr)])
        if tail:
            @pl.when(s == 0)
            def _():
                pltpu.sync_copy(acc.at[pl.ds(_NS * npr, tail)],
                                out_ref.at[pl.ds(c * n + _NS * npr, tail)])

    return body(ef_aug, dstr2, zer)


def _sc_final(ab4, src, dst, n, e):
    mesh = plsc.VectorSubcoreMesh(core_axis_name="c", subcore_axis_name="s")
    ept = e // (_NC * _NS)

    @functools.partial(
        pl.kernel,
        out_type=jax.ShapeDtypeStruct((2 * e,), jnp.float32),
        mesh=mesh,
        compiler_params=pltpu.CompilerParams(use_tc_tiling_on_sc=False, needs_layout_passes=False),
        scratch_types=[
            pltpu.VMEM((4 * n,), jnp.float32),
            pltpu.VMEM((ept,), jnp.int32),
            pltpu.VMEM((ept,), jnp.int32),
            pltpu.VMEM((2 * ept,), jnp.float32),
        ],
    )
    def body(ab_ref, src_ref, dst_ref, out_ref, ab_v, src_v, dst_v, out_v):
        c = lax.axis_index("c")
        s = lax.axis_index("s")
        w = c * _NS + s
        pltpu.sync_copy(ab_ref, ab_v)
        pltpu.sync_copy(src_ref.at[pl.ds(w * ept, ept)], src_v)
        pltpu.sync_copy(dst_ref.at[pl.ds(w * ept, ept)], dst_v)
        lanes = lax.iota(jnp.int32, 16)

        def step(i, carry):
            s16 = src_v[pl.ds(i * 16, 16)] * 4
            d16 = dst_v[pl.ds(i * 16, 16)] * 4
            a0 = plsc.load_gather(ab_v, [s16])
            a1 = plsc.load_gather(ab_v, [s16 + 1])
            b0 = plsc.load_gather(ab_v, [d16 + 2])
            b1 = plsc.load_gather(ab_v, [d16 + 3])
            rows = i * 32 + lanes * 2
            plsc.store_scatter(out_v, [rows], a0 + b0)
            plsc.store_scatter(out_v, [rows + 1], a1 + b1)
            return carry

        lax.fori_loop(0, ept // 16, step, 0)
        pltpu.sync_copy(out_v, out_ref.at[pl.ds(w * 2 * ept, 2 * ept)])

    return body(ab4, src, dst)


def kernel(nfeats, efeats, edge_index, Wm1, bm1, Wa1, ba1, Wm2, bm2, Wa2,
           ba2, Wm3, bm3, Wa3, ba3, Wp, bp):
    n = nfeats.shape[0]
    e = efeats.shape[0]
    din = nfeats.shape[2]
    edim = efeats.shape[2]
    emb = Wm1.shape[1]
    half = emb // 2
    hp = 80
    ncls = Wp.shape[1]

    src = edge_index[0]
    dst = edge_index[1]
    h0 = nfeats[:, 0, :]

    ef_aug = jnp.concatenate(
        [jnp.ones((e, 1), jnp.float32), efeats[:, 0, :],
         jnp.zeros((e, hp - 1 - edim), jnp.float32)], axis=1)

    def msg_weights(Wm, bm):
        dh = Wm.shape[0] - edim
        aug = jnp.concatenate(
            [bm[None, :], Wm[dh:], jnp.zeros((hp - 1 - edim, emb))], axis=0)
        return (_pad_cols(aug[:, :half], hp), _pad_cols(aug[:, half:], hp))

    def apply_weights(Wa, dh):
        waa = Wa[:dh]
        wab0 = _pad_rows(Wa[dh:dh + half], hp)
        wab1 = _pad_rows(Wa[dh + half:], hp)
        return waa, wab0, wab1

    def gather_table_w(Wm, dh):
        wa = Wm[:dh]
        return jnp.stack([_pad_cols(wa[:, :half], hp),
                          _pad_cols(wa[:, half:], hp)])

    wmb0_1, wmb1_1 = msg_weights(Wm1, bm1)
    wmb0_2, wmb1_2 = msg_weights(Wm2, bm2)
    wmb0_3, wmb1_3 = msg_weights(Wm3, bm3)
    waa1, wab0_1, wab1_1 = apply_weights(Wa1, din)
    waa2, wab0_2, wab1_2 = apply_weights(Wa2, emb)
    waa3, wab0_3, wab1_3 = apply_weights(Wa3, emb)
    wpre = gather_table_w(Wm1, din)
    wnext2 = gather_table_w(Wm2, emb)
    wnext3 = gather_table_w(Wm3, emb)
    wp_pad = _pad_cols(jnp.concatenate([Wp[:emb], Wp[emb:]], axis=1), 128)
    bp_pad = _pad_cols(bp[None, :], 128)

    eps = e // _NS
    cs = eps // _K
    ept = e // (_NC * _NS)
    ce = ept // _K
    srcr = (src.reshape(_NS, cs, _K)[None]
            + (jnp.arange(_NC, dtype=jnp.int32) * n)[:, None, None, None])
    dstr = dst.reshape(_NS, cs, _K)
    dstr2 = dst.reshape(_NC, _NS, ce, _K)
    zer = jnp.zeros((n, hp), jnp.float32)

    esum2 = _sc_esum(ef_aug, dstr2, zer, n, hp, ce)

    t1 = _tc_pre(h0, wpre, n, hp)
    s1 = _sc_scatter(t1, srcr, dstr, zer, n, hp, cs)
    h1, t2 = _tc_mid(h0, s1, esum2, wmb0_1, wmb1_1, waa1, wab0_1, wab1_1,
                     ba1[None, :], wnext2, n, hp, emb)
    s2 = _sc_scatter(t2, srcr, dstr, zer, n, hp, cs)
    h2, t3 = _tc_mid(h1, s2, esum2, wmb0_2, wmb1_2, waa2, wab0_2, wab1_2,
                     ba2[None, :], wnext3, n, hp, emb)
    s3 = _sc_scatter(t3, srcr, dstr, zer, n, hp, cs)
    ab = _tc_final(h2, s3, esum2, wmb0_3, wmb1_3, waa3, wab0_3, wab1_3,
                   ba3[None, :], wp_pad, bp_pad, n, hp, emb)

    ab4 = ab[:, :2 * ncls].reshape(2 * ncls * n)
    score = _sc_final(ab4, src, dst, n, e)
    return score.reshape(e, ncls)
